# Initial kernel scaffold; baseline (speedup 1.0000x reference)
import jax
import jax.numpy as jnp
from jax.experimental import pallas as pl


def kernel(embeddings, target, triplets):
    raise NotImplementedError("write your pallas kernel here")


# speedup vs baseline: 3.4338x; 3.4338x over previous
import functools

import jax
import jax.numpy as jnp
from jax import lax
from jax.experimental import pallas as pl
from jax.experimental.pallas import tpu as pltpu
from jax.experimental.pallas import tpu_sc as plsc

MARGIN = 1.0
NUM_TRIPLETS = 16384
EMB_ROWS = 4096
EMB_DIM = 128
LANES = 16
NC = 2
NS = 16
NW = NC * NS
T_PER_W = NUM_TRIPLETS // NW
CHUNK = 128
N_CHUNKS = T_PER_W // CHUNK
VPR = EMB_DIM // LANES


def _tec_body(emb_hbm, ai_hbm, pi_hbm, ni_hbm, out_hbm,
              idx_a, idx_p, idx_n, rows_a, rows_p, rows_n, tbuf, out_v,
              sem_a, sem_p, sem_n):
    wid = lax.axis_index("s") * NC + lax.axis_index("c")
    base = wid * T_PER_W
    lane = lax.iota(jnp.int32, LANES)

    def chunk_total(c, carry):
        off = base + c * CHUNK
        pltpu.sync_copy(ai_hbm.at[pl.ds(off, CHUNK)], idx_a)
        pltpu.sync_copy(pi_hbm.at[pl.ds(off, CHUNK)], idx_p)
        pltpu.sync_copy(ni_hbm.at[pl.ds(off, CHUNK)], idx_n)
        da = pltpu.async_copy(emb_hbm.at[idx_a], rows_a, sem_a)
        dp = pltpu.async_copy(emb_hbm.at[idx_p], rows_p, sem_p)
        dn = pltpu.async_copy(emb_hbm.at[idx_n], rows_n, sem_n)
        da.wait()
        dp.wait()
        dn.wait()

        def group(g, tot):
            def triplet(tt, _):
                t = g * LANES + tt
                acc = jnp.zeros((LANES,), jnp.float32)
                for j in range(VPR):
                    sl = pl.ds(j * LANES, LANES)
                    a = rows_a[t, sl]
                    p = rows_p[t, sl]
                    n = rows_n[t, sl]
                    dap = a - p
                    dan = a - n
                    acc = acc + (dap * dap - dan * dan)
                plsc.store_scatter(tbuf, [lane * LANES + tt], acc)
                return 0

            lax.fori_loop(0, LANES, triplet, 0)
            v = tbuf[pl.ds(0, LANES)]
            for j in range(1, LANES):
                v = v + tbuf[pl.ds(j * LANES, LANES)]
            return tot + jnp.maximum(v + MARGIN, 0.0)

        return lax.fori_loop(0, CHUNK // LANES, group, carry)

    total = lax.fori_loop(0, N_CHUNKS, chunk_total,
                          jnp.zeros((LANES,), jnp.float32))
    out_v[...] = total
    pltpu.sync_copy(out_v, out_hbm.at[wid])


@jax.jit
def _triplet_loss_sc(emb, ai, pi, ni):
    mesh = plsc.VectorSubcoreMesh(core_axis_name="c", subcore_axis_name="s")
    partials = pl.kernel(
        _tec_body,
        out_type=jax.ShapeDtypeStruct((NW, LANES), jnp.float32),
        mesh=mesh,
        compiler_params=pltpu.CompilerParams(needs_layout_passes=False),
        scratch_types=[
            pltpu.VMEM((CHUNK,), jnp.int32),
            pltpu.VMEM((CHUNK,), jnp.int32),
            pltpu.VMEM((CHUNK,), jnp.int32),
            pltpu.VMEM((CHUNK, EMB_DIM), jnp.float32),
            pltpu.VMEM((CHUNK, EMB_DIM), jnp.float32),
            pltpu.VMEM((CHUNK, EMB_DIM), jnp.float32),
            pltpu.VMEM((LANES * LANES,), jnp.float32),
            pltpu.VMEM((LANES,), jnp.float32),
            pltpu.SemaphoreType.DMA,
            pltpu.SemaphoreType.DMA,
            pltpu.SemaphoreType.DMA,
        ],
    )(emb, ai, pi, ni)
    return jnp.sum(partials) / jnp.float32(NUM_TRIPLETS)


def kernel(embeddings, target, triplets):
    del target
    trip = triplets.astype(jnp.int32)
    ai = trip[:, 0]
    pi = trip[:, 1]
    ni = trip[:, 2]
    mean = _triplet_loss_sc(embeddings, ai, pi, ni)
    return (mean, jnp.asarray(NUM_TRIPLETS, dtype=jnp.int32))


# speedup vs baseline: 4.0089x; 1.1675x over previous
import jax
import jax.numpy as jnp
from jax import lax
from jax.experimental import pallas as pl
from jax.experimental.pallas import tpu as pltpu
from jax.experimental.pallas import tpu_sc as plsc

MARGIN = 1.0
NUM_TRIPLETS = 16384
EMB_DIM = 128
LANES = 16
NC = 2
NS = 16
NW = NC * NS
T_PER_W = NUM_TRIPLETS // NW
CHUNK = 128
N_CHUNKS = T_PER_W // CHUNK
VPR = EMB_DIM // LANES
GROUPS = CHUNK // LANES


def _tec_body(emb_hbm, ai_hbm, pi_hbm, ni_hbm, out_hbm,
              idx_a, idx_p, idx_n, rows, tbuf, out_v, sems):
    wid = lax.axis_index("s") * NC + lax.axis_index("c")
    lane16 = lax.iota(jnp.int32, LANES) * LANES

    pltpu.sync_copy(ai_hbm.at[wid], idx_a)
    pltpu.sync_copy(pi_hbm.at[wid], idx_p)
    pltpu.sync_copy(ni_hbm.at[wid], idx_n)

    def issue(c):
        buf = c % 2
        return [
            pltpu.async_copy(emb_hbm.at[idx.at[c]], rows.at[buf, r], sems.at[buf])
            for r, idx in enumerate((idx_a, idx_p, idx_n))
        ]

    def compute(c, tot):
        buf = c % 2
        ra, rp, rn = rows.at[buf, 0], rows.at[buf, 1], rows.at[buf, 2]

        def group(g, tot):
            t0 = g * LANES
            for tt in range(LANES):
                t = t0 + tt
                acc = jnp.zeros((LANES,), jnp.float32)
                for j in range(VPR):
                    sl = pl.ds(j * LANES, LANES)
                    a = ra[t, sl]
                    p = rp[t, sl]
                    n = rn[t, sl]
                    dap = a - p
                    dan = a - n
                    acc = acc + (dap * dap - dan * dan)
                plsc.store_scatter(tbuf, [lane16 + tt], acc)
            v = tbuf[pl.ds(0, LANES)]
            for j in range(1, LANES):
                v = v + tbuf[pl.ds(j * LANES, LANES)]
            return tot + jnp.maximum(v + MARGIN, 0.0)

        return lax.fori_loop(0, GROUPS, group, tot)

    tot = jnp.zeros((LANES,), jnp.float32)
    descs = issue(0)
    for c in range(N_CHUNKS):
        for d in descs:
            d.wait()
        if c + 1 < N_CHUNKS:
            descs = issue(c + 1)
        tot = compute(c, tot)

    out_v[...] = tot
    pltpu.sync_copy(out_v, out_hbm.at[wid])


@jax.jit
def _triplet_loss_sc(emb, ai, pi, ni):
    mesh = plsc.VectorSubcoreMesh(core_axis_name="c", subcore_axis_name="s")
    partials = pl.kernel(
        _tec_body,
        out_type=jax.ShapeDtypeStruct((NW, LANES), jnp.float32),
        mesh=mesh,
        compiler_params=pltpu.CompilerParams(needs_layout_passes=False),
        scratch_types=[
            pltpu.VMEM((N_CHUNKS, CHUNK), jnp.int32),
            pltpu.VMEM((N_CHUNKS, CHUNK), jnp.int32),
            pltpu.VMEM((N_CHUNKS, CHUNK), jnp.int32),
            pltpu.VMEM((2, 3, CHUNK, EMB_DIM), jnp.float32),
            pltpu.VMEM((LANES * LANES,), jnp.float32),
            pltpu.VMEM((LANES,), jnp.float32),
            pltpu.SemaphoreType.DMA((2,)),
        ],
    )(emb, ai, pi, ni)
    return jnp.sum(partials) / jnp.float32(NUM_TRIPLETS)


def kernel(embeddings, target, triplets):
    del target
    trip = triplets.astype(jnp.int32)
    ai = trip[:, 0].reshape(NW, N_CHUNKS, CHUNK)
    pi = trip[:, 1].reshape(NW, N_CHUNKS, CHUNK)
    ni = trip[:, 2].reshape(NW, N_CHUNKS, CHUNK)
    mean = _triplet_loss_sc(embeddings, ai, pi, ni)
    return (mean, jnp.asarray(NUM_TRIPLETS, dtype=jnp.int32))
